# Initial kernel scaffold; baseline (speedup 1.0000x reference)
import jax
import jax.numpy as jnp
from jax.experimental import pallas as pl


def kernel(positions, bonds, W_msg, b_msg, W_upd, b_upd):
    raise NotImplementedError("write your pallas kernel here")


# speedup vs baseline: 227.2698x; 227.2698x over previous
import functools

import jax
import jax.numpy as jnp
from jax import lax
from jax.experimental import pallas as pl
from jax.experimental.pallas import tpu as pltpu
from jax.experimental.pallas import tpu_sc as plsc

N_ATOMS = 10000
BATCH = 8
ROW = 32
NC, NS = 2, 16
NW = NC * NS
GRP = 128
G = 79
EPW = G * GRP
E_PAD = NW * EPW
N_PAD = 10112
STRIPE = N_PAD // NS


def _segsum_sc(h_hbm, src_hbm, dst_hbm, zeros_hbm, out_hbm,
               src_v, dst_v, rows_v, zbuf_v, acc_s, sem):
    c = lax.axis_index("c")
    s = lax.axis_index("s")
    wid = c * NS + s
    pltpu.sync_copy(src_hbm.at[wid], src_v)
    pltpu.sync_copy(dst_hbm.at[wid], dst_v)
    pltpu.sync_copy(zeros_hbm.at[pl.ds(s * STRIPE, STRIPE)], zbuf_v)
    pltpu.sync_copy(zbuf_v, acc_s.at[pl.ds(s * STRIPE, STRIPE)])
    plsc.subcore_barrier()

    def body(g, _):
        pltpu.async_copy(h_hbm.at[src_v.at[g]], rows_v, sem).wait()
        pltpu.sync_copy(rows_v, acc_s.at[dst_v.at[g]], add=True)
        return _

    lax.fori_loop(0, G, body, None)
    plsc.subcore_barrier()
    pltpu.sync_copy(acc_s.at[pl.ds(s * STRIPE, STRIPE)],
                    out_hbm.at[c, pl.ds(s * STRIPE, STRIPE)])


@functools.cache
def _build_segsum():
    return pl.kernel(
        _segsum_sc,
        mesh=plsc.VectorSubcoreMesh(core_axis_name="c", subcore_axis_name="s"),
        compiler_params=pltpu.CompilerParams(use_tc_tiling_on_sc=False),
        out_type=jax.ShapeDtypeStruct((NC, N_PAD, ROW), jnp.float32),
        scratch_types=[
            pltpu.VMEM((G, GRP), jnp.int32),
            pltpu.VMEM((G, GRP), jnp.int32),
            pltpu.VMEM((GRP, ROW), jnp.float32),
            pltpu.VMEM((STRIPE, ROW), jnp.float32),
            pltpu.VMEM_SHARED((N_PAD, ROW), jnp.float32),
            pltpu.SemaphoreType.DMA,
        ],
    )


def _update_tc(h_ref, p_ref, t_ref, b_ref, o_ref):
    S = p_ref[0] + p_ref[1]
    q = 1.0 / jnp.maximum(S[:, 3:4], 1.0)
    o_ref[...] = (h_ref[...]
                  + jnp.dot(S * q, t_ref[...],
                            preferred_element_type=jnp.float32,
                            precision=jax.lax.Precision.HIGHEST)
                  + b_ref[...])


def _update(h, parts, t32, b32):
    return pl.pallas_call(
        _update_tc,
        out_shape=jax.ShapeDtypeStruct((N_PAD, ROW), jnp.float32),
    )(h, parts, t32, b32)


def kernel(positions, bonds, W_msg, b_msg, W_upd, b_upd):
    src = jnp.concatenate([bonds[:, 0], bonds[:, 1]])
    dst = jnp.concatenate([bonds[:, 1], bonds[:, 0]])
    pad = jnp.full((E_PAD - src.shape[0],), N_ATOMS, jnp.int32)
    src = jnp.concatenate([src, pad]).reshape(NW, G, GRP)
    dst = jnp.concatenate([dst, pad]).reshape(NW, G, GRP)

    M = W_upd @ W_msg
    v = W_upd @ b_msg
    t4 = jnp.zeros((4, 4), jnp.float32).at[:3, :3].set(M.T).at[3, :3].set(v)
    t32 = jnp.kron(jnp.eye(8, dtype=jnp.float32), t4)
    b32 = jnp.tile(jnp.concatenate([b_upd, jnp.zeros((1,), jnp.float32)]),
                   BATCH)[None, :]

    hp = jnp.concatenate(
        [positions.transpose(1, 0, 2),
         jnp.ones((N_ATOMS, BATCH, 1), jnp.float32)], axis=2
    ).reshape(N_ATOMS, ROW)
    h = jnp.zeros((N_PAD, ROW), jnp.float32).at[:N_ATOMS].set(hp)

    zeros = jnp.zeros((N_PAD, ROW), jnp.float32)
    segsum = _build_segsum()
    for _ in range(2):
        parts = segsum(h, src, dst, zeros)
        h = _update(h, parts, t32, b32)

    return h[:N_ATOMS].reshape(N_ATOMS, BATCH, 4)[:, :, :3].transpose(1, 0, 2)
